# Initial kernel scaffold; baseline (speedup 1.0000x reference)
import jax
import jax.numpy as jnp
from jax.experimental import pallas as pl


def kernel(x, lookup_dict):
    raise NotImplementedError("write your pallas kernel here")


# speedup vs baseline: 4.0840x; 4.0840x over previous
import functools

import jax
import jax.numpy as jnp
from jax import lax
from jax.experimental import pallas as pl
from jax.experimental.pallas import tpu as pltpu
from jax.experimental.pallas import tpu_sc as plsc

_NC = 2
_NS = 16
_NW = _NC * _NS
_CH = 128


def _gather_call(n_chunks, V, D):
    c_per_w = n_chunks // _NW
    mesh = plsc.VectorSubcoreMesh(core_axis_name="c", subcore_axis_name="s")

    @functools.partial(
        pl.kernel,
        mesh=mesh,
        compiler_params=pltpu.CompilerParams(use_tc_tiling_on_sc=False),
        out_type=jax.ShapeDtypeStruct((n_chunks * _CH, D), jnp.float32),
        scratch_types=[
            pltpu.VMEM((c_per_w * _CH,), jnp.int32),
            pltpu.VMEM((_CH, D), jnp.float32),
            pltpu.SemaphoreType.DMA,
        ],
    )
    def k(idx_hbm, tab_hbm, out_hbm, idx_v, rows_v, sem):
        wid = lax.axis_index("s") * _NC + lax.axis_index("c")
        base = wid * c_per_w
        pltpu.sync_copy(idx_hbm.at[pl.ds(base * _CH, c_per_w * _CH)], idx_v)

        def body(j, carry):
            idx_chunk = idx_v.at[pl.ds(j * _CH, _CH)]
            pltpu.async_copy(tab_hbm.at[idx_chunk], rows_v, sem).wait()
            pltpu.sync_copy(rows_v, out_hbm.at[pl.ds((base + j) * _CH, _CH)])
            return carry

        lax.fori_loop(0, c_per_w, body, 0)

    return k


def kernel(x, lookup_dict):
    B, H = x.shape
    V, D = lookup_dict.shape
    n = B * H
    n_chunks = n // _CH
    assert n % (_CH * _NW) == 0
    idx_flat = x.reshape(n).astype(jnp.int32)
    out = _gather_call(n_chunks, V, D)(idx_flat, lookup_dict)
    return out.reshape(B, H, D)


# speedup vs baseline: 4.6771x; 1.1452x over previous
import functools

import jax
import jax.numpy as jnp
from jax import lax
from jax.experimental import pallas as pl
from jax.experimental.pallas import tpu as pltpu
from jax.experimental.pallas import tpu_sc as plsc

_NC = 2
_NS = 16
_NW = _NC * _NS
_CH = 128


_NB = 5


def _gather_call(n_chunks, V, D):
    c_per_w = n_chunks // _NW
    n_groups = c_per_w // _NB
    mesh = plsc.VectorSubcoreMesh(core_axis_name="c", subcore_axis_name="s")

    @functools.partial(
        pl.kernel,
        mesh=mesh,
        compiler_params=pltpu.CompilerParams(use_tc_tiling_on_sc=False),
        out_type=jax.ShapeDtypeStruct((n_chunks * _CH, D), jnp.float32),
        scratch_types=[
            pltpu.VMEM((c_per_w * _CH,), jnp.int32),
            pltpu.VMEM((_NB, _CH, D), jnp.float32),
            pltpu.SemaphoreType.DMA,
            pltpu.SemaphoreType.DMA,
        ],
    )
    def k(idx_hbm, tab_hbm, out_hbm, idx_v, rows_v, gsem, ssem):
        wid = lax.axis_index("s") * _NC + lax.axis_index("c")
        base = wid * c_per_w
        pltpu.sync_copy(idx_hbm.at[pl.ds(base * _CH, c_per_w * _CH)], idx_v)

        def start_gather(j, b):
            idx_chunk = idx_v.at[pl.ds(j * _CH, _CH)]
            pltpu.make_async_copy(tab_hbm.at[idx_chunk], rows_v.at[b], gsem).start()

        def wait_gather(b):
            pltpu.make_async_copy(tab_hbm.at[idx_v.at[pl.ds(0, _CH)]],
                                  rows_v.at[b], gsem).wait()

        def start_store(j, b):
            pltpu.make_async_copy(
                rows_v.at[b], out_hbm.at[pl.ds((base + j) * _CH, _CH)], ssem
            ).start()

        def wait_store(b):
            pltpu.make_async_copy(
                rows_v.at[b], out_hbm.at[pl.ds(base * _CH, _CH)], ssem
            ).wait()

        for b in range(_NB):
            start_gather(b, b)

        def body(g, carry):
            for b in range(_NB):
                j = g * _NB + b
                wait_gather(b)
                start_store(j, b)
            for b in range(_NB):
                jn = (g + 1) * _NB + b
                wait_store(b)
                start_gather(jn, b)
            return carry

        lax.fori_loop(0, n_groups - 1, body, 0)

        g = n_groups - 1
        for b in range(_NB):
            wait_gather(b)
            start_store(g * _NB + b, b)
        for b in range(_NB):
            wait_store(b)

    return k


def kernel(x, lookup_dict):
    B, H = x.shape
    V, D = lookup_dict.shape
    n = B * H
    n_chunks = n // _CH
    assert n % (_CH * _NW) == 0
    idx_flat = x.reshape(n).astype(jnp.int32)
    out = _gather_call(n_chunks, V, D)(idx_flat, lookup_dict)
    return out.reshape(B, H, D)
